# Initial kernel scaffold; baseline (speedup 1.0000x reference)
import jax
import jax.numpy as jnp
from jax.experimental import pallas as pl


def kernel(inputs, hx_k, gconv_w, gconv_b, W, b, R, att_w, att_b, proj_w, proj_b, support):
    raise NotImplementedError("write your pallas kernel here")


# speedup vs baseline: 2.1738x; 2.1738x over previous
import jax
import jax.numpy as jnp
from jax.experimental import pallas as pl

N = 1024
B = 32
D = 64
PRE_K = 4


def _cell_kernel(in_ref, hx_ref, g0_ref, g3_ref, g2_ref, gb_ref, w_ref,
                 bias_ref, r_ref, aw_ref, ab_ref, pw_ref, pb_ref, s_ref,
                 y_ref, hxo_ref):
    h = hx_ref[...]
    aw = aw_ref[...]
    r = r_ref[...]

    ck = jnp.sum(r * aw[None], axis=(1, 2))
    hl = jnp.sum(h[0] * aw[None], axis=(1, 2))
    logits = hl + ck + ab_ref[0]
    m = jnp.max(logits)
    e = jnp.exp(logits - m)
    wts = e / jnp.sum(e)

    att = wts[0] * (h[0, 0] + r[0])
    att = att + wts[1] * (h[0, 1] + r[1])
    att = att + wts[2] * (h[0, 2] + r[2])
    att = att + wts[3] * (h[0, 3] + r[3])

    h3 = h[0, 3]
    h2 = h[0, 2]
    x_in = in_ref[...][0]
    a = jnp.dot(h3, g3_ref[...], preferred_element_type=jnp.float32)
    a = a + jnp.dot(h2, g2_ref[...], preferred_element_type=jnp.float32)
    a = a + x_in * g0_ref[...][None, :]

    a0 = a[:, 0:D]
    a1 = a[:, D:2 * D]
    a2 = a[:, 2 * D:3 * D]

    s = s_ref[...]
    u = jnp.dot(s, a2, preferred_element_type=jnp.float32)
    v = a1 + 2.0 * u
    t = jnp.dot(s, v, preferred_element_type=jnp.float32)
    g = a0 - a2 + t + gb_ref[...][None, :]

    conv = jnp.where(g >= 0, g, 0.01 * g)
    out = jnp.dot(conv, w_ref[...], preferred_element_type=jnp.float32)
    out = out + bias_ref[...] + att

    hxo_ref[:, 0:3] = h[:, 1:4]
    hxo_ref[0, 3] = out
    y_ref[...] = jnp.sum(out * pw_ref[...][None, :], axis=1,
                         keepdims=True)[None] + pb_ref[0]


def kernel(inputs, hx_k, gconv_w, gconv_b, W, b, R, att_w, att_b, proj_w,
           proj_b, support):
    gcat = gconv_w.reshape(1 + 2 * D, 3, D).reshape(1 + 2 * D, 3 * D)
    g0 = gcat[0]
    g3 = gcat[1:1 + D]
    g2 = gcat[1 + D:]
    awm = att_w.reshape(N, D)
    pw = proj_w.reshape(D)
    hx0 = hx_k[0]

    y, hx_new = pl.pallas_call(
        _cell_kernel,
        grid=(B,),
        in_specs=[
            pl.BlockSpec((1, N, 1), lambda i: (i, 0, 0)),
            pl.BlockSpec((1, PRE_K, N, D), lambda i: (i, 0, 0, 0)),
            pl.BlockSpec((3 * D,), lambda i: (0,)),
            pl.BlockSpec((D, 3 * D), lambda i: (0, 0)),
            pl.BlockSpec((D, 3 * D), lambda i: (0, 0)),
            pl.BlockSpec((D,), lambda i: (0,)),
            pl.BlockSpec((D, D), lambda i: (0, 0)),
            pl.BlockSpec((N, D), lambda i: (0, 0)),
            pl.BlockSpec((PRE_K, N, D), lambda i: (0, 0, 0)),
            pl.BlockSpec((N, D), lambda i: (0, 0)),
            pl.BlockSpec((1,), lambda i: (0,)),
            pl.BlockSpec((D,), lambda i: (0,)),
            pl.BlockSpec((1,), lambda i: (0,)),
            pl.BlockSpec((N, N), lambda i: (0, 0)),
        ],
        out_specs=[
            pl.BlockSpec((1, N, 1), lambda i: (i, 0, 0)),
            pl.BlockSpec((1, PRE_K, N, D), lambda i: (i, 0, 0, 0)),
        ],
        out_shape=[
            jax.ShapeDtypeStruct((B, N, 1), jnp.float32),
            jax.ShapeDtypeStruct((B, PRE_K, N, D), jnp.float32),
        ],
    )(inputs[:, :, None], hx0, g0, g3, g2, gconv_b, W, b, R, awm, att_b, pw,
      proj_b, support)
    return y.reshape(B, N), hx_new[None]


# speedup vs baseline: 2.6581x; 1.2227x over previous
import jax
import jax.numpy as jnp
from jax.experimental import pallas as pl

N = 1024
B = 32
D = 64
PRE_K = 4
BB = 4


def _cell_kernel(in_ref, hx_ref, g0_ref, g3_ref, g2_ref, gb_ref, w_ref,
                 bias_ref, r_ref, aw_ref, ab_ref, pw_ref, pb_ref, s_ref,
                 y_ref, hxo_ref):
    h = hx_ref[...]
    aw = aw_ref[...]
    r = r_ref[...]

    ck = jnp.sum(r * aw[None], axis=(1, 2))
    hl = jnp.sum(h * aw[None, None], axis=(2, 3))
    logits = hl + ck[None, :] + ab_ref[0]
    m = jnp.max(logits, axis=1, keepdims=True)
    e = jnp.exp(logits - m)
    wts = e / jnp.sum(e, axis=1, keepdims=True)

    h3 = h[:, 3].reshape(BB * N, D)
    h2 = h[:, 2].reshape(BB * N, D)
    a = jnp.dot(h3, g3_ref[...], preferred_element_type=jnp.float32)
    a = a + jnp.dot(h2, g2_ref[...], preferred_element_type=jnp.float32)
    a = a.reshape(BB, N, 3 * D)
    a = a + in_ref[...] * g0_ref[...][None, None, :]

    a0 = jnp.concatenate([a[i, :, 0:D] for i in range(BB)], axis=1)
    a1 = jnp.concatenate([a[i, :, D:2 * D] for i in range(BB)], axis=1)
    a2 = jnp.concatenate([a[i, :, 2 * D:3 * D] for i in range(BB)], axis=1)

    s = s_ref[...]
    u = jnp.dot(s, a2, preferred_element_type=jnp.float32)
    v = a1 + 2.0 * u
    t = jnp.dot(s, v, preferred_element_type=jnp.float32)
    gb4 = jnp.concatenate([gb_ref[...]] * BB)
    g = a0 - a2 + t + gb4[None, :]

    conv = jnp.where(g >= 0, g, 0.01 * g)
    hxo_ref[:, 0:3] = h[:, 1:4]
    pw = pw_ref[...]
    for i in range(BB):
        out_i = jnp.dot(conv[:, i * D:(i + 1) * D], w_ref[...],
                        preferred_element_type=jnp.float32)
        att_i = wts[i, 0] * (h[i, 0] + r[0])
        att_i = att_i + wts[i, 1] * (h[i, 1] + r[1])
        att_i = att_i + wts[i, 2] * (h[i, 2] + r[2])
        att_i = att_i + wts[i, 3] * (h[i, 3] + r[3])
        out_i = out_i + bias_ref[...] + att_i
        hxo_ref[i, 3] = out_i
        y_ref[i] = jnp.sum(out_i * pw[None, :], axis=1,
                           keepdims=True) + pb_ref[0]


def kernel(inputs, hx_k, gconv_w, gconv_b, W, b, R, att_w, att_b, proj_w,
           proj_b, support):
    gcat = gconv_w.reshape(1 + 2 * D, 3, D).reshape(1 + 2 * D, 3 * D)
    g0 = gcat[0]
    g3 = gcat[1:1 + D]
    g2 = gcat[1 + D:]
    awm = att_w.reshape(N, D)
    pw = proj_w.reshape(D)
    hx0 = hx_k[0]

    y, hx_new = pl.pallas_call(
        _cell_kernel,
        grid=(B // BB,),
        in_specs=[
            pl.BlockSpec((BB, N, 1), lambda i: (i, 0, 0)),
            pl.BlockSpec((BB, PRE_K, N, D), lambda i: (i, 0, 0, 0)),
            pl.BlockSpec((3 * D,), lambda i: (0,)),
            pl.BlockSpec((D, 3 * D), lambda i: (0, 0)),
            pl.BlockSpec((D, 3 * D), lambda i: (0, 0)),
            pl.BlockSpec((D,), lambda i: (0,)),
            pl.BlockSpec((D, D), lambda i: (0, 0)),
            pl.BlockSpec((N, D), lambda i: (0, 0)),
            pl.BlockSpec((PRE_K, N, D), lambda i: (0, 0, 0)),
            pl.BlockSpec((N, D), lambda i: (0, 0)),
            pl.BlockSpec((1,), lambda i: (0,)),
            pl.BlockSpec((D,), lambda i: (0,)),
            pl.BlockSpec((1,), lambda i: (0,)),
            pl.BlockSpec((N, N), lambda i: (0, 0)),
        ],
        out_specs=[
            pl.BlockSpec((BB, N, 1), lambda i: (i, 0, 0)),
            pl.BlockSpec((BB, PRE_K, N, D), lambda i: (i, 0, 0, 0)),
        ],
        out_shape=[
            jax.ShapeDtypeStruct((B, N, 1), jnp.float32),
            jax.ShapeDtypeStruct((B, PRE_K, N, D), jnp.float32),
        ],
    )(inputs[:, :, None], hx0, g0, g3, g2, gconv_b, W, b, R, awm, att_b, pw,
      proj_b, support)
    return y.reshape(B, N), hx_new[None]
